# baseline (speedup 1.0000x reference)
import functools

import jax
import jax.numpy as jnp
from jax import lax
from jax.experimental import pallas as pl
from jax.experimental.pallas import tpu as pltpu
from jax.experimental.pallas import tpu_sc as plsc

_NC = 2
_NS = 16
_NW = _NC * _NS
_IW = 128


def _sc_gather(uid, iid, user_mf, item_mf, user_mlp, item_mlp):
    n_rows, _ = uid.shape
    b = n_rows * _IW
    ch = n_rows // _NW
    bw = ch * _IW
    d = user_mf.shape[1]

    mesh = plsc.VectorSubcoreMesh(
        core_axis_name="c", subcore_axis_name="s",
        num_cores=_NC, num_subcores=_NS,
    )

    out_t = [jax.ShapeDtypeStruct((b, d), jnp.float32)] * 4

    @functools.partial(
        pl.kernel,
        out_type=out_t,
        mesh=mesh,
        compiler_params=pltpu.CompilerParams(use_tc_tiling_on_sc=False),
        scratch_types=[
            pltpu.VMEM((ch, _IW), jnp.int32),
            pltpu.VMEM((ch, _IW), jnp.int32),
            pltpu.VMEM((bw, d), jnp.float32),
            pltpu.VMEM((bw, d), jnp.float32),
            pltpu.VMEM((bw, d), jnp.float32),
            pltpu.VMEM((bw, d), jnp.float32),
            pltpu.SemaphoreType.DMA,
        ],
    )
    def gather_kernel(uid_h, iid_h, umf_h, imf_h, umlp_h, imlp_h,
                      o_umf, o_imf, o_umlp, o_imlp,
                      uidx, iidx, g_umf, g_imf, g_umlp, g_imlp, sem):
        c = lax.axis_index("c")
        s = lax.axis_index("s")
        w = s * _NC + c
        base = w * ch
        pltpu.sync_copy(uid_h.at[pl.ds(base, ch)], uidx)
        pltpu.sync_copy(iid_h.at[pl.ds(base, ch)], iidx)
        copies = []
        for j in range(ch):
            r = pl.ds(j * _IW, _IW)
            copies.append(pltpu.async_copy(umf_h.at[uidx.at[j]], g_umf.at[r], sem))
            copies.append(pltpu.async_copy(imf_h.at[iidx.at[j]], g_imf.at[r], sem))
            copies.append(pltpu.async_copy(umlp_h.at[uidx.at[j]], g_umlp.at[r], sem))
            copies.append(pltpu.async_copy(imlp_h.at[iidx.at[j]], g_imlp.at[r], sem))
        for cp in copies:
            cp.wait()
        rows = pl.ds(w * bw, bw)
        pltpu.sync_copy(g_umf, o_umf.at[rows])
        pltpu.sync_copy(g_imf, o_imf.at[rows])
        pltpu.sync_copy(g_umlp, o_umlp.at[rows])
        pltpu.sync_copy(g_imlp, o_imlp.at[rows])

    return gather_kernel(uid, iid, user_mf, item_mf, user_mlp, item_mlp)


def _mlp_body(umf, imf, umlp, imlp, w1u, w1i, b1, w2, b2, w3, b3,
              wpm, wph, bp, out):
    f32 = jnp.float32
    h = jnp.dot(umlp[...], w1u[...], preferred_element_type=f32)
    h += jnp.dot(imlp[...], w1i[...], preferred_element_type=f32)
    h = jnp.maximum(h + b1[...], 0.0)
    h = jnp.maximum(jnp.dot(h, w2[...], preferred_element_type=f32) + b2[...], 0.0)
    h = jnp.maximum(jnp.dot(h, w3[...], preferred_element_type=f32) + b3[...], 0.0)
    logits = jnp.dot(umf[...] * imf[...], wpm[...], preferred_element_type=f32)
    logits += jnp.dot(h, wph[...], preferred_element_type=f32) + bp[...]
    out[...] = jax.nn.sigmoid(logits)


def _tc_mlp(u_mf, i_mf, u_mlp, i_mlp, W1, b1, W2, b2, W3, b3, Wp, bp):
    b = u_mf.shape[0]
    d = u_mf.shape[1]
    h1, h2, h3 = W1.shape[1], W2.shape[1], W3.shape[1]
    bt = 4096
    grid = (b // bt,)

    w1u, w1i = W1[:d], W1[d:]
    wpm, wph = Wp[:d], Wp[d:]
    b1r, b2r, b3r, bpr = (b1.reshape(1, -1), b2.reshape(1, -1),
                          b3.reshape(1, -1), bp.reshape(1, -1))

    row_spec = pl.BlockSpec((bt, d), lambda i: (i, 0))
    full = lambda shape: pl.BlockSpec(shape, lambda i: (0, 0))

    return pl.pallas_call(
        _mlp_body,
        grid=grid,
        in_specs=[
            row_spec, row_spec, row_spec, row_spec,
            full((d, h1)), full((d, h1)), full((1, h1)),
            full((h1, h2)), full((1, h2)),
            full((h2, h3)), full((1, h3)),
            full((d, 1)), full((h3, 1)), full((1, 1)),
        ],
        out_specs=pl.BlockSpec((bt, 1), lambda i: (i, 0)),
        out_shape=jax.ShapeDtypeStruct((b, 1), jnp.float32),
    )(u_mf, i_mf, u_mlp, i_mlp, w1u, w1i, b1r, W2, b2r, W3, b3r,
      wpm, wph, bpr)


def kernel(user_id, item_id, user_mf, item_mf, user_mlp, item_mlp,
           W1, b1, W2, b2, W3, b3, Wp, bp):
    b = user_id.shape[0]
    uid = user_id.astype(jnp.int32).reshape(b // _IW, _IW)
    iid = item_id.astype(jnp.int32).reshape(b // _IW, _IW)
    u_mf, i_mf, u_mlp, i_mlp = _sc_gather(
        uid, iid, user_mf, item_mf, user_mlp, item_mlp)
    return _tc_mlp(u_mf, i_mf, u_mlp, i_mlp,
                   W1, b1, W2, b2, W3, b3, Wp, bp)


# speedup vs baseline: 6.9375x; 6.9375x over previous
import functools

import jax
import jax.numpy as jnp
from jax import lax
from jax.experimental import pallas as pl
from jax.experimental.pallas import tpu as pltpu
from jax.experimental.pallas import tpu_sc as plsc

_NC = 2
_NS = 16
_NW = _NC * _NS
_LANES = 16
_TW = 128
_GRP = 16
_Q = 1
_NB = 8
_LEAD = 7


def _sc_gather_t(uid, iid, umf_t, imf_t, umlp_t, imlp_t):
    b = uid.shape[0]
    bw = b // _NW
    d = umf_t.shape[0]
    ngrp = bw // _GRP

    mesh = plsc.VectorSubcoreMesh(
        core_axis_name="c", subcore_axis_name="s",
        num_cores=_NC, num_subcores=_NS,
    )

    out_t = [jax.ShapeDtypeStruct((d, b), jnp.float32)] * 4

    slab = pltpu.VMEM((d, _Q * _TW), jnp.float32)

    @functools.partial(
        pl.kernel,
        out_type=out_t,
        mesh=mesh,
        compiler_params=pltpu.CompilerParams(
            disable_bounds_checks=True, needs_layout_passes=False),
        scratch_types=[
            pltpu.VMEM((bw,), jnp.int32),
            pltpu.VMEM((bw,), jnp.int32),
        ] + [slab] * (4 * _NB) + [
            pltpu.VMEM((d, bw), jnp.float32),
            pltpu.VMEM((d, bw), jnp.float32),
            pltpu.VMEM((d, bw), jnp.float32),
            pltpu.VMEM((d, bw), jnp.float32),
        ] + [pltpu.SemaphoreType.DMA] * _NB,
    )
    def gather_kernel(uid_h, iid_h, umf_h, imf_h, umlp_h, imlp_h,
                      o_umf, o_imf, o_umlp, o_imlp, *scr):
        uidx, iidx = scr[0], scr[1]
        slab_refs = scr[2:2 + 4 * _NB]
        g_umf, g_imf, g_umlp, g_imlp = scr[2 + 4 * _NB:6 + 4 * _NB]
        sems = scr[6 + 4 * _NB:]
        ci = lax.axis_index("c")
        si = lax.axis_index("s")
        w = si * _NC + ci
        pltpu.sync_copy(uid_h.at[pl.ds(w * bw, bw)], uidx)
        pltpu.sync_copy(iid_h.at[pl.ds(w * bw, bw)], iidx)

        iota = lax.iota(jnp.int32, _LANES)

        banks = tuple(
            ((slab_refs[4 * i], slab_refs[4 * i + 2],
              slab_refs[4 * i + 1], slab_refs[4 * i + 3]), sems[i])
            for i in range(_NB))
        nb = _NB
        lead = _LEAD
        tabs = (umf_h, umlp_h, imf_h, imlp_h)

        def fire_quad(q, bank, rb_uv, rb_iv):
            slabs, sem = banks[bank]
            for k in range(_Q):
                lane = _Q * q + k
                rbu = pl.multiple_of(rb_uv[lane], _TW)
                rbi = pl.multiple_of(rb_iv[lane], _TW)
                for t, (slab_r, tab) in enumerate(zip(slabs, tabs)):
                    rb = rbu if t < 2 else rbi
                    pltpu.async_copy(tab.at[:, pl.ds(rb, _TW)],
                                     slab_r.at[:, pl.ds(k * _TW, _TW)], sem)

        def drain_extract(bank, lane0, pos0, rlo_uv, rlo_iv):
            slabs, sem = banks[bank]
            for slab_r in slabs:
                pltpu.make_async_copy(
                    umf_h.at[:, pl.ds(0, _Q * _TW)], slab_r, sem).wait()
            gs = (g_umf, g_umlp, g_imf, g_imlp)
            for k in range(_Q):
                lane = lane0 + k
                gcols = jnp.full((_LANES,), pos0 + k, jnp.int32)
                for t, (slab_r, g) in enumerate(zip(slabs, gs)):
                    rlo = rlo_uv[lane] if t < 2 else rlo_iv[lane]
                    cols = jnp.full((_LANES,), rlo + k * _TW, jnp.int32)
                    vals = plsc.load_gather(slab_r, [iota, cols])
                    plsc.store_scatter(g, [iota, gcols], vals)

        def load_group(g):
            ruv = uidx[pl.ds(g * _GRP, _GRP)]
            riv = iidx[pl.ds(g * _GRP, _GRP)]
            rlo_uv = lax.rem(ruv, _TW)
            rlo_iv = lax.rem(riv, _TW)
            return rlo_uv, rlo_iv, ruv - rlo_uv, riv - rlo_iv

        nquad = _GRP // _Q

        rlo_uv0, rlo_iv0, rb_uv0, rb_iv0 = load_group(0)
        for q in range(lead):
            fire_quad(q, q % nb, rb_uv0, rb_iv0)
        for q in range(lead, nquad):
            fire_quad(q, q % nb, rb_uv0, rb_iv0)
            drain_extract((q - lead) % nb, _Q * (q - lead), _Q * (q - lead),
                          rlo_uv0, rlo_iv0)

        def group(g, carry):
            rlo_uv_p, rlo_iv_p = carry
            rlo_uv, rlo_iv, rb_uv, rb_iv = load_group(g)
            base = g * _GRP
            for q in range(lead):
                fire_quad(q, q % nb, rb_uv, rb_iv)
                pq = nquad - lead + q
                drain_extract(pq % nb, _Q * pq, base - _GRP + _Q * pq,
                              rlo_uv_p, rlo_iv_p)
            for q in range(lead, nquad):
                fire_quad(q, q % nb, rb_uv, rb_iv)
                drain_extract((q - lead) % nb, _Q * (q - lead),
                              base + _Q * (q - lead), rlo_uv, rlo_iv)
            return (rlo_uv, rlo_iv)

        rlo_uv_l, rlo_iv_l = lax.fori_loop(
            1, ngrp, group, (rlo_uv0, rlo_iv0))
        for q in range(lead):
            pq = nquad - lead + q
            drain_extract(pq % nb, _Q * pq, (ngrp - 1) * _GRP + _Q * pq,
                          rlo_uv_l, rlo_iv_l)

        wcols = pl.ds(pl.multiple_of(w * bw, _TW), bw)
        for gbuf, out in ((g_umf, o_umf), (g_imf, o_imf),
                         (g_umlp, o_umlp), (g_imlp, o_imlp)):
            pltpu.sync_copy(gbuf, out.at[:, wcols])

    return gather_kernel(uid, iid, umf_t, imf_t, umlp_t, imlp_t)


def _mlp_body(umf, imf, umlp, imlp, w1u, w1i, b1, w2, b2, w3, b3,
              wpm, wph, bp, out):
    f32 = jnp.float32
    h = jnp.dot(w1u[...], umlp[...], preferred_element_type=f32)
    h += jnp.dot(w1i[...], imlp[...], preferred_element_type=f32)
    h = jnp.maximum(h + b1[...], 0.0)
    h = jnp.maximum(jnp.dot(w2[...], h, preferred_element_type=f32) + b2[...], 0.0)
    h = jnp.maximum(jnp.dot(w3[...], h, preferred_element_type=f32) + b3[...], 0.0)
    logits = jnp.dot(wpm[...], umf[...] * imf[...], preferred_element_type=f32)
    logits += jnp.dot(wph[...], h, preferred_element_type=f32) + bp[...]
    out[...] = jax.nn.sigmoid(logits)


def _tc_mlp_t(u_mf, i_mf, u_mlp, i_mlp, W1, b1, W2, b2, W3, b3, Wp, bp):
    d, b = u_mf.shape
    h1, h2, h3 = W1.shape[1], W2.shape[1], W3.shape[1]
    bt = 4096
    grid = (b // bt,)

    w1u_t, w1i_t = W1[:d].T, W1[d:].T
    wpm_t, wph_t = Wp[:d].T, Wp[d:].T
    w2_t, w3_t = W2.T, W3.T
    b1c, b2c, b3c, bpc = (b1.reshape(-1, 1), b2.reshape(-1, 1),
                          b3.reshape(-1, 1), bp.reshape(-1, 1))

    col_spec = pl.BlockSpec((d, bt), lambda i: (0, i))
    full = lambda shape: pl.BlockSpec(shape, lambda i: (0, 0))

    return pl.pallas_call(
        _mlp_body,
        grid=grid,
        in_specs=[
            col_spec, col_spec, col_spec, col_spec,
            full((h1, d)), full((h1, d)), full((h1, 1)),
            full((h2, h1)), full((h2, 1)),
            full((h3, h2)), full((h3, 1)),
            full((1, d)), full((1, h3)), full((1, 1)),
        ],
        out_specs=pl.BlockSpec((1, bt), lambda i: (0, i)),
        out_shape=jax.ShapeDtypeStruct((1, b), jnp.float32),
    )(u_mf, i_mf, u_mlp, i_mlp, w1u_t, w1i_t, b1c, w2_t, b2c, w3_t, b3c,
      wpm_t, wph_t, bpc)


def kernel(user_id, item_id, user_mf, item_mf, user_mlp, item_mlp,
           W1, b1, W2, b2, W3, b3, Wp, bp):
    b = user_id.shape[0]
    uid = user_id.astype(jnp.int32)
    iid = item_id.astype(jnp.int32)
    u_mf, i_mf, u_mlp, i_mlp = _sc_gather_t(
        uid, iid, user_mf.T, item_mf.T, user_mlp.T, item_mlp.T)
    out_t = _tc_mlp_t(u_mf, i_mf, u_mlp, i_mlp,
                      W1, b1, W2, b2, W3, b3, Wp, bp)
    return out_t.T
